# baseline (speedup 1.0000x reference)
import jax
import jax.numpy as jnp
from jax import lax
from jax.experimental import pallas as pl
from jax.experimental.pallas import tpu as pltpu
from jax.experimental.pallas import tpu_sc as plsc

N = 10000
E = 320000
D = 128
C = 128
HALF = 64
NTILE = 16
NPAD = 10240
RPT = NPAD // NTILE
CH = 80
ET = E // NTILE
NCH = ET // CH
U = 2
NOUT = NCH // U


def _rsqrt_newton(d):
    s = d * 0.25 + 1.0
    for _ in range(12):
        s = 0.5 * (s + d / s)
    return 1.0 / s


def _sc_body(g_hbm, h_hbm, out0_hbm, out1_hbm, acc, deg,
             tbuf, dbuf, ones, sidxb, didxb, rows, isem, gsm, ssem, hsem):
    c = lax.axis_index("c")
    s = lax.axis_index("s")
    r0 = s * RPT
    e0 = s * ET

    zv = jnp.zeros((16,), jnp.float32)
    ov = jnp.ones((16,), jnp.float32)

    def _zero_tbuf():
        @pl.loop(0, RPT)
        def _(i):
            for k in range(HALF // 16):
                tbuf[i, pl.ds(16 * k, 16)] = zv

    _zero_tbuf()
    pltpu.sync_copy(tbuf, acc.at[pl.ds(r0, RPT)])

    @pl.loop(0, RPT)
    def _(i):
        dbuf[i, :] = zv

    pltpu.sync_copy(dbuf, deg.at[pl.ds(r0, RPT)])

    @pl.loop(0, CH)
    def _(i):
        ones[i, :] = ov

    @pl.when(s < NTILE - 1)
    def _():
        pltpu.async_copy(
            h_hbm.at[pl.ds(r0, RPT), pl.ds(HALF * c, HALF)], tbuf, hsem)

    @pl.when(s == NTILE - 1)
    def _():
        last = N - (NTILE - 1) * RPT
        pltpu.async_copy(
            h_hbm.at[pl.ds((NTILE - 1) * RPT, last), pl.ds(HALF * c, HALF)],
            tbuf.at[pl.ds(0, last)], hsem)

    plsc.subcore_barrier()


    def _idx_fire(gi, u, slot_par, dst_too):
        off = e0 + (gi * U + u) * CH
        row = slot_par * U + u
        if dst_too:
            pltpu.async_copy(g_hbm.at[0, pl.ds(off, CH)], sidxb.at[row],
                             isem[u])
        pltpu.async_copy(g_hbm.at[1, pl.ds(off, CH)], didxb.at[row], isem[u])

    def _idx_wait(gi, u, slot_par, dst_too):
        off = e0 + (gi * U + u) * CH
        row = slot_par * U + u
        if dst_too:
            pltpu.make_async_copy(g_hbm.at[0, pl.ds(off, CH)],
                                  sidxb.at[row], isem[u]).wait()
        pltpu.make_async_copy(g_hbm.at[1, pl.ds(off, CH)],
                              didxb.at[row], isem[u]).wait()

    def _edge_pass(scatter_src_is_ones, table_hbm):
        with_gather = not scatter_src_is_ones

        for u in range(U):
            _idx_fire(0, u, 0, with_gather)

        @pl.loop(0, NOUT)
        def _(g):
            par = g % 2
            nxt = (g + 1) % 2

            for u in range(U):
                @pl.when(g > 0)
                def _():
                    prow = nxt * U + u
                    if scatter_src_is_ones:
                        pltpu.make_async_copy(
                            ones, deg.at[didxb.at[prow]], ssem[u]).wait()
                    else:
                        pltpu.make_async_copy(
                            rows[u], acc.at[didxb.at[prow]], ssem[u]).wait()

                _idx_wait(g, u, par, with_gather)
                if with_gather:
                    pltpu.async_copy(
                        table_hbm.at[sidxb.at[par * U + u]], rows[u], gsm[u])
                else:
                    pltpu.async_copy(
                        ones, deg.at[didxb.at[par * U + u]], ssem[u],
                        add=True)

            @pl.when(g < NOUT - 1)
            def _():
                for u in range(U):
                    _idx_fire(g + 1, u, nxt, with_gather)

            if with_gather:
                for u in range(U):
                    pltpu.make_async_copy(
                        table_hbm.at[sidxb.at[par * U + u]], rows[u],
                        gsm[u]).wait()
                    pltpu.async_copy(
                        rows[u], acc.at[didxb.at[par * U + u]], ssem[u],
                        add=True)

        lrow = ((NOUT - 1) % 2) * U
        for u in range(U):
            if scatter_src_is_ones:
                pltpu.make_async_copy(
                    ones, deg.at[didxb.at[lrow + u]], ssem[u]).wait()
            else:
                pltpu.make_async_copy(
                    rows[u], acc.at[didxb.at[lrow + u]], ssem[u]).wait()

    _edge_pass(scatter_src_is_ones=True, table_hbm=None)
    plsc.subcore_barrier()

    pltpu.sync_copy(deg.at[pl.ds(r0, RPT)], dbuf)

    @pl.loop(0, RPT)
    def _(j):
        d = jnp.maximum(dbuf[j, :], 1.0)
        dbuf[j, :] = _rsqrt_newton(d)

    @pl.when(s < NTILE - 1)
    def _():
        pltpu.make_async_copy(
            h_hbm.at[pl.ds(r0, RPT), pl.ds(HALF * c, HALF)], tbuf,
            hsem).wait()

    @pl.when(s == NTILE - 1)
    def _():
        last = N - (NTILE - 1) * RPT
        pltpu.make_async_copy(
            h_hbm.at[pl.ds((NTILE - 1) * RPT, last), pl.ds(HALF * c, HALF)],
            tbuf.at[pl.ds(0, last)], hsem).wait()

    def _scale_rows(power2):
        @pl.loop(0, RPT)
        def _(j):
            nj = dbuf[j, :]
            njp = nj * nj if power2 else nj
            for k in range(HALF // 16):
                tbuf[j, pl.ds(16 * k, 16)] = tbuf[j, pl.ds(16 * k, 16)] * njp

    def _write_table():
        @pl.when(c == 0)
        def _():
            pltpu.sync_copy(tbuf, out0_hbm.at[pl.ds(r0, RPT)])

        @pl.when(c == 1)
        def _():
            pltpu.sync_copy(tbuf, out1_hbm.at[pl.ds(r0, RPT)])

    _scale_rows(False)
    _write_table()
    plsc.subcore_barrier()

    def _hop():
        @pl.when(c == 0)
        def _():
            _edge_pass(scatter_src_is_ones=False, table_hbm=out0_hbm)

        @pl.when(c == 1)
        def _():
            _edge_pass(scatter_src_is_ones=False, table_hbm=out1_hbm)

    _hop()
    plsc.subcore_barrier()

    pltpu.sync_copy(acc.at[pl.ds(r0, RPT)], tbuf)
    _scale_rows(True)
    _write_table()
    _zero_tbuf()
    pltpu.sync_copy(tbuf, acc.at[pl.ds(r0, RPT)])
    plsc.subcore_barrier()

    _hop()
    plsc.subcore_barrier()

    pltpu.sync_copy(acc.at[pl.ds(r0, RPT)], tbuf)
    _scale_rows(False)
    _write_table()


_sc_propagate = pl.kernel(
    _sc_body,
    out_type=(
        jax.ShapeDtypeStruct((NPAD, HALF), jnp.float32),
        jax.ShapeDtypeStruct((NPAD, HALF), jnp.float32),
    ),
    mesh=plsc.VectorSubcoreMesh(core_axis_name="c", subcore_axis_name="s"),
    compiler_params=pltpu.CompilerParams(use_tc_tiling_on_sc=False),
    scratch_types=[
        pltpu.VMEM_SHARED((NPAD, HALF), jnp.float32),
        pltpu.VMEM_SHARED((NPAD, 16), jnp.float32),
        pltpu.VMEM((RPT, HALF), jnp.float32),
        pltpu.VMEM((RPT, 16), jnp.float32),
        pltpu.VMEM((CH, 16), jnp.float32),
        pltpu.VMEM((2 * U, CH), jnp.int32),
        pltpu.VMEM((2 * U, CH), jnp.int32),
        [pltpu.VMEM((CH, HALF), jnp.float32)] * U,
        [pltpu.SemaphoreType.DMA] * U,
        [pltpu.SemaphoreType.DMA] * U,
        [pltpu.SemaphoreType.DMA] * U,
        pltpu.SemaphoreType.DMA,
    ],
)


def _mm_body(f0_ref, f1_ref, w0_ref, w1_ref, b_ref, o_ref):
    o_ref[...] = (
        jnp.dot(f0_ref[...], w0_ref[...], preferred_element_type=jnp.float32)
        + jnp.dot(f1_ref[...], w1_ref[...], preferred_element_type=jnp.float32)
        + b_ref[...]
    )


BM = 1024

_tc_matmul = pl.pallas_call(
    _mm_body,
    grid=(NPAD // BM,),
    in_specs=[
        pl.BlockSpec((BM, HALF), lambda i: (i, 0)),
        pl.BlockSpec((BM, HALF), lambda i: (i, 0)),
        pl.BlockSpec((HALF, C), lambda i: (0, 0)),
        pl.BlockSpec((HALF, C), lambda i: (0, 0)),
        pl.BlockSpec((1, C), lambda i: (0, 0)),
    ],
    out_specs=pl.BlockSpec((BM, C), lambda i: (i, 0)),
    out_shape=jax.ShapeDtypeStruct((NPAD, C), jnp.float32),
)


@jax.jit
def kernel(g, h, e, snorm_n, snorm_e, W, b):
    f0, f1 = _sc_propagate(g, h)
    out = _tc_matmul(f0, f1, W[:HALF], W[HALF:], b.reshape(1, C))
    return out[:N]


# speedup vs baseline: 1.4475x; 1.4475x over previous
import jax
import jax.numpy as jnp
from jax import lax
from jax.experimental import pallas as pl
from jax.experimental.pallas import tpu as pltpu
from jax.experimental.pallas import tpu_sc as plsc

N = 10000
E = 320000
D = 128
C = 128
HALF = 64
NTILE = 16
NPAD = 10240
RPT = NPAD // NTILE
QRPT = RPT // 4
CH = 80
ET = E // NTILE
NCH = ET // CH
U = 10
NOUT = NCH // U


def _rsqrt_newton(d):
    s = d * 0.25 + 1.0
    for _ in range(12):
        s = 0.5 * (s + d / s)
    return 1.0 / s


def _sc_body(g_hbm, h_hbm, out0_hbm, out1_hbm, acc, deg,
             tbuf, dbuf, ones, sidxb, didxb, rows, isem, gsm, ssem, hsem):
    c = lax.axis_index("c")
    s = lax.axis_index("s")
    r0 = s * RPT
    e0 = s * ET

    zv = jnp.zeros((16,), jnp.float32)
    ov = jnp.ones((16,), jnp.float32)

    def _zero_tbuf():
        @pl.loop(0, QRPT)
        def _(i):
            for k in range(HALF // 16):
                tbuf[i, pl.ds(16 * k, 16)] = zv

    _zero_tbuf()
    for q in range(4):
        pltpu.sync_copy(tbuf, acc.at[pl.ds(r0 + q * QRPT, QRPT)])

    @pl.loop(0, RPT)
    def _(i):
        dbuf[i, :] = zv

    pltpu.sync_copy(dbuf, deg.at[pl.ds(r0, RPT)])

    @pl.loop(0, CH)
    def _(i):
        ones[i, :] = ov

    plsc.subcore_barrier()


    def _idx_fire(gi, u, slot_par, dst_too):
        off = e0 + (gi * U + u) * CH
        row = slot_par * U + u
        if dst_too:
            pltpu.async_copy(g_hbm.at[0, pl.ds(off, CH)], sidxb.at[row],
                             isem[u])
        pltpu.async_copy(g_hbm.at[1, pl.ds(off, CH)], didxb.at[row], isem[u])

    def _idx_wait(gi, u, slot_par, dst_too):
        off = e0 + (gi * U + u) * CH
        row = slot_par * U + u
        if dst_too:
            pltpu.make_async_copy(g_hbm.at[0, pl.ds(off, CH)],
                                  sidxb.at[row], isem[u]).wait()
        pltpu.make_async_copy(g_hbm.at[1, pl.ds(off, CH)],
                              didxb.at[row], isem[u]).wait()

    def _edge_pass(scatter_src_is_ones, table_hbm):
        with_gather = not scatter_src_is_ones

        for u in range(U):
            _idx_fire(0, u, 0, with_gather)

        @pl.loop(0, NOUT)
        def _(g):
            par = g % 2
            nxt = (g + 1) % 2

            for u in range(U):
                @pl.when(g > 0)
                def _():
                    prow = nxt * U + u
                    if scatter_src_is_ones:
                        pltpu.make_async_copy(
                            ones, deg.at[didxb.at[prow]], ssem[u]).wait()
                    else:
                        pltpu.make_async_copy(
                            rows[u], acc.at[didxb.at[prow]], ssem[u]).wait()

                _idx_wait(g, u, par, with_gather)
                if with_gather:
                    pltpu.async_copy(
                        table_hbm.at[sidxb.at[par * U + u]], rows[u], gsm[u])
                else:
                    pltpu.async_copy(
                        ones, deg.at[didxb.at[par * U + u]], ssem[u],
                        add=True)

            @pl.when(g < NOUT - 1)
            def _():
                for u in range(U):
                    _idx_fire(g + 1, u, nxt, with_gather)

            if with_gather:
                for u in range(U):
                    pltpu.make_async_copy(
                        table_hbm.at[sidxb.at[par * U + u]], rows[u],
                        gsm[u]).wait()
                    pltpu.async_copy(
                        rows[u], acc.at[didxb.at[par * U + u]], ssem[u],
                        add=True)

        lrow = ((NOUT - 1) % 2) * U
        for u in range(U):
            if scatter_src_is_ones:
                pltpu.make_async_copy(
                    ones, deg.at[didxb.at[lrow + u]], ssem[u]).wait()
            else:
                pltpu.make_async_copy(
                    rows[u], acc.at[didxb.at[lrow + u]], ssem[u]).wait()

    _edge_pass(scatter_src_is_ones=True, table_hbm=None)
    plsc.subcore_barrier()

    pltpu.sync_copy(deg.at[pl.ds(r0, RPT)], dbuf)

    @pl.loop(0, RPT)
    def _(j):
        d = jnp.maximum(dbuf[j, :], 1.0)
        dbuf[j, :] = _rsqrt_newton(d)

    def _scale_rows(power2, q):
        @pl.loop(0, QRPT)
        def _(j):
            nj = dbuf[q * QRPT + j, :]
            njp = nj * nj if power2 else nj
            for k in range(HALF // 16):
                tbuf[j, pl.ds(16 * k, 16)] = tbuf[j, pl.ds(16 * k, 16)] * njp

    def _write_table(q, nrows=QRPT):
        @pl.when(c == 0)
        def _():
            pltpu.sync_copy(tbuf.at[pl.ds(0, nrows)],
                            out0_hbm.at[pl.ds(r0 + q * QRPT, nrows)])

        @pl.when(c == 1)
        def _():
            pltpu.sync_copy(tbuf.at[pl.ds(0, nrows)],
                            out1_hbm.at[pl.ds(r0 + q * QRPT, nrows)])

    for q in range(4):
        base = (NTILE - 1) * RPT + q * QRPT
        lastq = min(QRPT, max(0, N - base))

        @pl.when(s < NTILE - 1)
        def _():
            pltpu.sync_copy(
                h_hbm.at[pl.ds(r0 + q * QRPT, QRPT), pl.ds(HALF * c, HALF)],
                tbuf)
            _scale_rows(False, q)
            _write_table(q)

        if lastq > 0:
            @pl.when(s == NTILE - 1)
            def _():
                pltpu.sync_copy(
                    h_hbm.at[pl.ds(base, lastq), pl.ds(HALF * c, HALF)],
                    tbuf.at[pl.ds(0, lastq)])
                _scale_rows(False, q)
                _write_table(q, lastq)

    plsc.subcore_barrier()

    def _hop():
        @pl.when(c == 0)
        def _():
            _edge_pass(scatter_src_is_ones=False, table_hbm=out0_hbm)

        @pl.when(c == 1)
        def _():
            _edge_pass(scatter_src_is_ones=False, table_hbm=out1_hbm)

    def _rescale(power2):
        for q in range(4):
            pltpu.sync_copy(acc.at[pl.ds(r0 + q * QRPT, QRPT)], tbuf)
            _scale_rows(power2, q)
            _write_table(q)

    _hop()
    plsc.subcore_barrier()

    _rescale(True)
    _zero_tbuf()
    for q in range(4):
        pltpu.sync_copy(tbuf, acc.at[pl.ds(r0 + q * QRPT, QRPT)])
    plsc.subcore_barrier()

    _hop()
    plsc.subcore_barrier()

    _rescale(False)


_sc_propagate = pl.kernel(
    _sc_body,
    out_type=(
        jax.ShapeDtypeStruct((NPAD, HALF), jnp.float32),
        jax.ShapeDtypeStruct((NPAD, HALF), jnp.float32),
    ),
    mesh=plsc.VectorSubcoreMesh(core_axis_name="c", subcore_axis_name="s"),
    compiler_params=pltpu.CompilerParams(use_tc_tiling_on_sc=False),
    scratch_types=[
        pltpu.VMEM_SHARED((NPAD, HALF), jnp.float32),
        pltpu.VMEM_SHARED((NPAD, 16), jnp.float32),
        pltpu.VMEM((QRPT, HALF), jnp.float32),
        pltpu.VMEM((RPT, 16), jnp.float32),
        pltpu.VMEM((CH, 16), jnp.float32),
        pltpu.VMEM((2 * U, CH), jnp.int32),
        pltpu.VMEM((2 * U, CH), jnp.int32),
        [pltpu.VMEM((CH, HALF), jnp.float32)] * U,
        [pltpu.SemaphoreType.DMA] * U,
        [pltpu.SemaphoreType.DMA] * U,
        [pltpu.SemaphoreType.DMA] * U,
        pltpu.SemaphoreType.DMA,
    ],
)


def _mm_body(f0_ref, f1_ref, w0_ref, w1_ref, b_ref, o_ref):
    o_ref[...] = (
        jnp.dot(f0_ref[...], w0_ref[...], preferred_element_type=jnp.float32)
        + jnp.dot(f1_ref[...], w1_ref[...], preferred_element_type=jnp.float32)
        + b_ref[...]
    )


BM = 1024

_tc_matmul = pl.pallas_call(
    _mm_body,
    grid=(NPAD // BM,),
    in_specs=[
        pl.BlockSpec((BM, HALF), lambda i: (i, 0)),
        pl.BlockSpec((BM, HALF), lambda i: (i, 0)),
        pl.BlockSpec((HALF, C), lambda i: (0, 0)),
        pl.BlockSpec((HALF, C), lambda i: (0, 0)),
        pl.BlockSpec((1, C), lambda i: (0, 0)),
    ],
    out_specs=pl.BlockSpec((BM, C), lambda i: (i, 0)),
    out_shape=jax.ShapeDtypeStruct((NPAD, C), jnp.float32),
)


@jax.jit
def kernel(g, h, e, snorm_n, snorm_e, W, b):
    f0, f1 = _sc_propagate(g, h)
    out = _tc_matmul(f0, f1, W[:HALF], W[HALF:], b.reshape(1, C))
    return out[:N]


# speedup vs baseline: 1.5506x; 1.0712x over previous
import jax
import jax.numpy as jnp
from jax import lax
from jax.experimental import pallas as pl
from jax.experimental.pallas import tpu as pltpu
from jax.experimental.pallas import tpu_sc as plsc

N = 10000
E = 320000
D = 128
C = 128
HALF = 64
NTILE = 16
NPAD = 10240
RPT = NPAD // NTILE
QRPT = RPT // 4
CH = 128
ET = E // NTILE
NCH = ET // CH
TAIL = ET - NCH * CH
U = 4
NOUT = NCH // U


def _rsqrt_newton(d):
    s = d * 0.015625 + 16.0
    for _ in range(8):
        s = 0.5 * (s + d / s)
    return 1.0 / s


def _sc_body(g_hbm, h_hbm, out0_hbm, out1_hbm, acc, deg,
             tbuf, dbuf, ones, sidxb, didxb, tsidx, tdidx, rows,
             isem, gsm, ssem, hsem):
    c = lax.axis_index("c")
    s = lax.axis_index("s")
    r0 = s * RPT
    e0 = s * ET

    zv = jnp.zeros((16,), jnp.float32)
    ov = jnp.ones((16,), jnp.float32)

    def _zero_tbuf():
        @pl.loop(0, QRPT)
        def _(i):
            for k in range(HALF // 16):
                tbuf[i, pl.ds(16 * k, 16)] = zv

    _zero_tbuf()
    for q in range(4):
        pltpu.sync_copy(tbuf, acc.at[pl.ds(r0 + q * QRPT, QRPT)])

    @pl.loop(0, RPT)
    def _(i):
        dbuf[i, :] = zv

    pltpu.sync_copy(dbuf, deg.at[pl.ds(r0, RPT)])

    @pl.loop(0, CH)
    def _(i):
        ones[i, :] = ov

    plsc.subcore_barrier()


    def _idx_fire(gi, u, slot_par, dst_too):
        off = e0 + (gi * U + u) * CH
        row = slot_par * U + u
        if dst_too:
            pltpu.async_copy(g_hbm.at[0, pl.ds(off, CH)], sidxb.at[row],
                             isem[u])
        pltpu.async_copy(g_hbm.at[1, pl.ds(off, CH)], didxb.at[row], isem[u])

    def _idx_wait(gi, u, slot_par, dst_too):
        off = e0 + (gi * U + u) * CH
        row = slot_par * U + u
        if dst_too:
            pltpu.make_async_copy(g_hbm.at[0, pl.ds(off, CH)],
                                  sidxb.at[row], isem[u]).wait()
        pltpu.make_async_copy(g_hbm.at[1, pl.ds(off, CH)],
                              didxb.at[row], isem[u]).wait()

    def _edge_pass(scatter_src_is_ones, table_hbm):
        with_gather = not scatter_src_is_ones

        for u in range(U):
            _idx_fire(0, u, 0, with_gather)

        @pl.loop(0, NOUT)
        def _(g):
            par = g % 2
            nxt = (g + 1) % 2

            for u in range(U):
                @pl.when(g > 0)
                def _():
                    prow = nxt * U + u
                    if scatter_src_is_ones:
                        pltpu.make_async_copy(
                            ones, deg.at[didxb.at[prow]], ssem[u]).wait()
                    else:
                        pltpu.make_async_copy(
                            rows[u], acc.at[didxb.at[prow]], ssem[u]).wait()

                _idx_wait(g, u, par, with_gather)
                if with_gather:
                    pltpu.async_copy(
                        table_hbm.at[sidxb.at[par * U + u]], rows[u], gsm[u])
                else:
                    pltpu.async_copy(
                        ones, deg.at[didxb.at[par * U + u]], ssem[u],
                        add=True)

            @pl.when(g < NOUT - 1)
            def _():
                for u in range(U):
                    _idx_fire(g + 1, u, nxt, with_gather)

            if with_gather:
                for u in range(U):
                    pltpu.make_async_copy(
                        table_hbm.at[sidxb.at[par * U + u]], rows[u],
                        gsm[u]).wait()
                    pltpu.async_copy(
                        rows[u], acc.at[didxb.at[par * U + u]], ssem[u],
                        add=True)

        lrow = ((NOUT - 1) % 2) * U
        for u in range(U):
            if scatter_src_is_ones:
                pltpu.make_async_copy(
                    ones, deg.at[didxb.at[lrow + u]], ssem[u]).wait()
            else:
                pltpu.make_async_copy(
                    rows[u], acc.at[didxb.at[lrow + u]], ssem[u]).wait()

        toff = e0 + NCH * CH
        if with_gather:
            pltpu.sync_copy(g_hbm.at[0, pl.ds(toff, TAIL)], tsidx)
        pltpu.sync_copy(g_hbm.at[1, pl.ds(toff, TAIL)], tdidx)
        if with_gather:
            pltpu.async_copy(table_hbm.at[tsidx],
                             rows[0].at[pl.ds(0, TAIL)], gsm[0]).wait()
            pltpu.sync_copy(rows[0].at[pl.ds(0, TAIL)],
                            acc.at[tdidx], add=True)
        else:
            pltpu.sync_copy(ones.at[pl.ds(0, TAIL)],
                            deg.at[tdidx], add=True)

    _edge_pass(scatter_src_is_ones=True, table_hbm=None)
    plsc.subcore_barrier()

    pltpu.sync_copy(deg.at[pl.ds(r0, RPT)], dbuf)

    @pl.loop(0, RPT)
    def _(j):
        d = jnp.maximum(dbuf[j, :], 1.0)
        dbuf[j, :] = _rsqrt_newton(d)

    def _scale_rows(power2, q):
        @pl.loop(0, QRPT)
        def _(j):
            nj = dbuf[q * QRPT + j, :]
            njp = nj * nj if power2 else nj
            for k in range(HALF // 16):
                tbuf[j, pl.ds(16 * k, 16)] = tbuf[j, pl.ds(16 * k, 16)] * njp

    def _write_table(q, nrows=QRPT):
        @pl.when(c == 0)
        def _():
            pltpu.sync_copy(tbuf.at[pl.ds(0, nrows)],
                            out0_hbm.at[pl.ds(r0 + q * QRPT, nrows)])

        @pl.when(c == 1)
        def _():
            pltpu.sync_copy(tbuf.at[pl.ds(0, nrows)],
                            out1_hbm.at[pl.ds(r0 + q * QRPT, nrows)])

    for q in range(4):
        base = (NTILE - 1) * RPT + q * QRPT
        lastq = min(QRPT, max(0, N - base))

        @pl.when(s < NTILE - 1)
        def _():
            pltpu.sync_copy(
                h_hbm.at[pl.ds(r0 + q * QRPT, QRPT), pl.ds(HALF * c, HALF)],
                tbuf)
            _scale_rows(False, q)
            _write_table(q)

        if lastq > 0:
            @pl.when(s == NTILE - 1)
            def _():
                pltpu.sync_copy(
                    h_hbm.at[pl.ds(base, lastq), pl.ds(HALF * c, HALF)],
                    tbuf.at[pl.ds(0, lastq)])
                _scale_rows(False, q)
                _write_table(q, lastq)

    plsc.subcore_barrier()

    def _hop():
        @pl.when(c == 0)
        def _():
            _edge_pass(scatter_src_is_ones=False, table_hbm=out0_hbm)

        @pl.when(c == 1)
        def _():
            _edge_pass(scatter_src_is_ones=False, table_hbm=out1_hbm)

    def _rescale(power2):
        for q in range(4):
            pltpu.sync_copy(acc.at[pl.ds(r0 + q * QRPT, QRPT)], tbuf)
            _scale_rows(power2, q)
            _write_table(q)

    _hop()
    plsc.subcore_barrier()

    _rescale(True)
    _zero_tbuf()
    for q in range(4):
        pltpu.sync_copy(tbuf, acc.at[pl.ds(r0 + q * QRPT, QRPT)])
    plsc.subcore_barrier()

    _hop()
    plsc.subcore_barrier()

    _rescale(False)


_sc_propagate = pl.kernel(
    _sc_body,
    out_type=(
        jax.ShapeDtypeStruct((NPAD, HALF), jnp.float32),
        jax.ShapeDtypeStruct((NPAD, HALF), jnp.float32),
    ),
    mesh=plsc.VectorSubcoreMesh(core_axis_name="c", subcore_axis_name="s"),
    compiler_params=pltpu.CompilerParams(use_tc_tiling_on_sc=False),
    scratch_types=[
        pltpu.VMEM_SHARED((NPAD, HALF), jnp.float32),
        pltpu.VMEM_SHARED((NPAD, 16), jnp.float32),
        pltpu.VMEM((QRPT, HALF), jnp.float32),
        pltpu.VMEM((RPT, 16), jnp.float32),
        pltpu.VMEM((CH, 16), jnp.float32),
        pltpu.VMEM((2 * U, CH), jnp.int32),
        pltpu.VMEM((2 * U, CH), jnp.int32),
        pltpu.VMEM((TAIL,), jnp.int32),
        pltpu.VMEM((TAIL,), jnp.int32),
        [pltpu.VMEM((CH, HALF), jnp.float32)] * U,
        [pltpu.SemaphoreType.DMA] * U,
        [pltpu.SemaphoreType.DMA] * U,
        [pltpu.SemaphoreType.DMA] * U,
        pltpu.SemaphoreType.DMA,
    ],
)


def _mm_body(f0_ref, f1_ref, w0_ref, w1_ref, b_ref, o_ref):
    o_ref[...] = (
        jnp.dot(f0_ref[...], w0_ref[...], preferred_element_type=jnp.float32)
        + jnp.dot(f1_ref[...], w1_ref[...], preferred_element_type=jnp.float32)
        + b_ref[...]
    )


BM = 1024

_tc_matmul = pl.pallas_call(
    _mm_body,
    grid=(NPAD // BM,),
    in_specs=[
        pl.BlockSpec((BM, HALF), lambda i: (i, 0)),
        pl.BlockSpec((BM, HALF), lambda i: (i, 0)),
        pl.BlockSpec((HALF, C), lambda i: (0, 0)),
        pl.BlockSpec((HALF, C), lambda i: (0, 0)),
        pl.BlockSpec((1, C), lambda i: (0, 0)),
    ],
    out_specs=pl.BlockSpec((BM, C), lambda i: (i, 0)),
    out_shape=jax.ShapeDtypeStruct((NPAD, C), jnp.float32),
)


@jax.jit
def kernel(g, h, e, snorm_n, snorm_e, W, b):
    f0, f1 = _sc_propagate(g, h)
    out = _tc_matmul(f0, f1, W[:HALF], W[HALF:], b.reshape(1, C))
    return out[:N]


# speedup vs baseline: 1.6506x; 1.0645x over previous
import jax
import jax.numpy as jnp
from jax import lax
from jax.experimental import pallas as pl
from jax.experimental.pallas import tpu as pltpu
from jax.experimental.pallas import tpu_sc as plsc

N = 10000
E = 320000
D = 128
C = 128
HALF = 64
NTILE = 16
NPAD = 10240
RPT = NPAD // NTILE
QRPT = RPT // 4
CH = 128
ET = E // NTILE
NCH = ET // CH
TAIL = ET - NCH * CH
U = 6
NOUT = NCH // U
BR = 128
NB = RPT // BR


def _rsqrt_newton(d):
    s = d * 0.015625 + 16.0
    for _ in range(8):
        s = 0.5 * (s + d / s)
    return 1.0 / s


def _last_tile_rows(b):
    return min(BR, max(0, N - ((NTILE - 1) * RPT + b * BR)))


def _sc_body(g_hbm, h_hbm, out0_hbm, out1_hbm, acc, deg,
             tbuf, dbuf, ones, sidxb, didxb, tsidx, tdidx, rows,
             isem, gsm, ssem, hsem, wsem):
    c = lax.axis_index("c")
    s = lax.axis_index("s")
    r0 = s * RPT
    e0 = s * ET

    zv = jnp.zeros((16,), jnp.float32)
    ov = jnp.ones((16,), jnp.float32)

    def _zero_tbuf():
        @pl.loop(0, QRPT)
        def _(i):
            for k in range(HALF // 16):
                tbuf[i, pl.ds(16 * k, 16)] = zv

    def _zero_acc_slice():
        _zero_tbuf()
        for q in range(4):
            pltpu.sync_copy(tbuf, acc.at[pl.ds(r0 + q * QRPT, QRPT)])

    _zero_acc_slice()

    @pl.loop(0, RPT)
    def _(i):
        dbuf[i, :] = zv

    pltpu.sync_copy(dbuf, deg.at[pl.ds(r0, RPT)])

    @pl.loop(0, CH)
    def _(i):
        ones[i, :] = ov

    def _h_copies(fire):
        for b in range(NB):
            lastb = _last_tile_rows(b)

            @pl.when(s < NTILE - 1)
            def _():
                d = pltpu.make_async_copy(
                    h_hbm.at[pl.ds(r0 + b * BR, BR), pl.ds(HALF * c, HALF)],
                    rows[b], hsem)
                d.start() if fire else d.wait()

            if lastb > 0:
                @pl.when(s == NTILE - 1)
                def _():
                    sbase = (NTILE - 1) * RPT + b * BR
                    d = pltpu.make_async_copy(
                        h_hbm.at[pl.ds(sbase, lastb), pl.ds(HALF * c, HALF)],
                        rows[b].at[pl.ds(0, lastb)], hsem)
                    d.start() if fire else d.wait()

    _h_copies(fire=True)
    plsc.subcore_barrier()


    def _idx_fire(gi, u, slot_par, dst_too):
        off = e0 + (gi * U + u) * CH
        row = slot_par * U + u
        if dst_too:
            pltpu.async_copy(g_hbm.at[0, pl.ds(off, CH)], sidxb.at[row],
                             isem[u])
        pltpu.async_copy(g_hbm.at[1, pl.ds(off, CH)], didxb.at[row], isem[u])

    def _idx_wait(gi, u, slot_par, dst_too):
        off = e0 + (gi * U + u) * CH
        row = slot_par * U + u
        if dst_too:
            pltpu.make_async_copy(g_hbm.at[0, pl.ds(off, CH)],
                                  sidxb.at[row], isem[u]).wait()
        pltpu.make_async_copy(g_hbm.at[1, pl.ds(off, CH)],
                              didxb.at[row], isem[u]).wait()

    def _edge_pass(scatter_src_is_ones, table_hbm):
        with_gather = not scatter_src_is_ones

        for u in range(U):
            _idx_fire(0, u, 0, with_gather)

        @pl.loop(0, NOUT)
        def _(g):
            par = g % 2
            nxt = (g + 1) % 2

            for u in range(U):
                @pl.when(g > 0)
                def _():
                    prow = nxt * U + u
                    if scatter_src_is_ones:
                        pltpu.make_async_copy(
                            ones, deg.at[didxb.at[prow]], ssem[u]).wait()
                    else:
                        pltpu.make_async_copy(
                            rows[u], acc.at[didxb.at[prow]], ssem[u]).wait()

                _idx_wait(g, u, par, with_gather)
                if with_gather:
                    pltpu.async_copy(
                        table_hbm.at[sidxb.at[par * U + u]], rows[u], gsm[u])
                else:
                    pltpu.async_copy(
                        ones, deg.at[didxb.at[par * U + u]], ssem[u],
                        add=True)

            @pl.when(g < NOUT - 1)
            def _():
                for u in range(U):
                    _idx_fire(g + 1, u, nxt, with_gather)

            if with_gather:
                for u in range(U):
                    pltpu.make_async_copy(
                        table_hbm.at[sidxb.at[par * U + u]], rows[u],
                        gsm[u]).wait()
                    pltpu.async_copy(
                        rows[u], acc.at[didxb.at[par * U + u]], ssem[u],
                        add=True)

        lrow = ((NOUT - 1) % 2) * U
        for u in range(U):
            if scatter_src_is_ones:
                pltpu.make_async_copy(
                    ones, deg.at[didxb.at[lrow + u]], ssem[u]).wait()
            else:
                pltpu.make_async_copy(
                    rows[u], acc.at[didxb.at[lrow + u]], ssem[u]).wait()

        toff = e0 + NCH * CH
        if with_gather:
            pltpu.sync_copy(g_hbm.at[0, pl.ds(toff, TAIL)], tsidx)
        pltpu.sync_copy(g_hbm.at[1, pl.ds(toff, TAIL)], tdidx)
        if with_gather:
            pltpu.async_copy(table_hbm.at[tsidx],
                             rows[0].at[pl.ds(0, TAIL)], gsm[0]).wait()
            pltpu.sync_copy(rows[0].at[pl.ds(0, TAIL)],
                            acc.at[tdidx], add=True)
        else:
            pltpu.sync_copy(ones.at[pl.ds(0, TAIL)],
                            deg.at[tdidx], add=True)

    _edge_pass(scatter_src_is_ones=True, table_hbm=None)
    plsc.subcore_barrier()

    pltpu.sync_copy(deg.at[pl.ds(r0, RPT)], dbuf)

    @pl.loop(0, RPT)
    def _(j):
        d = jnp.maximum(dbuf[j, :], 1.0)
        dbuf[j, :] = _rsqrt_newton(d)

    def _scale_block(b, power2):
        @pl.loop(0, BR)
        def _(j):
            nj = dbuf[b * BR + j, :]
            njp = nj * nj if power2 else nj
            for k in range(HALF // 16):
                rows[b][j, pl.ds(16 * k, 16)] = (
                    rows[b][j, pl.ds(16 * k, 16)] * njp)

    def _write_block(b, nrows=BR):
        for cc, t in ((0, out0_hbm), (1, out1_hbm)):
            @pl.when(c == cc)
            def _():
                pltpu.make_async_copy(
                    rows[b].at[pl.ds(0, nrows)],
                    t.at[pl.ds(r0 + b * BR, nrows)], wsem).start()

    def _drain_writes(blocks_rows):
        for b, nrows in blocks_rows:
            for cc, t in ((0, out0_hbm), (1, out1_hbm)):
                @pl.when(c == cc)
                def _():
                    pltpu.make_async_copy(
                        rows[b].at[pl.ds(0, nrows)],
                        t.at[pl.ds(r0 + b * BR, nrows)], wsem).wait()

    _h_copies(fire=False)
    full_blocks = [(b, BR) for b in range(NB)]
    last_blocks = [(b, _last_tile_rows(b)) for b in range(NB)
                   if _last_tile_rows(b) > 0]
    for b in range(NB):
        lastb = _last_tile_rows(b)

        @pl.when(s < NTILE - 1)
        def _():
            _scale_block(b, False)
            _write_block(b)

        if lastb > 0:
            @pl.when(s == NTILE - 1)
            def _():
                _scale_block(b, False)
                _write_block(b, lastb)

    @pl.when(s < NTILE - 1)
    def _():
        _drain_writes(full_blocks)

    @pl.when(s == NTILE - 1)
    def _():
        _drain_writes(last_blocks)

    plsc.subcore_barrier()

    def _hop():
        @pl.when(c == 0)
        def _():
            _edge_pass(scatter_src_is_ones=False, table_hbm=out0_hbm)

        @pl.when(c == 1)
        def _():
            _edge_pass(scatter_src_is_ones=False, table_hbm=out1_hbm)

    def _rescale(power2):
        for b in range(NB):
            pltpu.async_copy(acc.at[pl.ds(r0 + b * BR, BR)], rows[b], hsem)
        for b in range(NB):
            pltpu.make_async_copy(
                acc.at[pl.ds(r0 + b * BR, BR)], rows[b], hsem).wait()
            _scale_block(b, power2)
            _write_block(b)
        _drain_writes(full_blocks)

    _hop()
    plsc.subcore_barrier()

    _rescale(True)
    _zero_acc_slice()
    plsc.subcore_barrier()

    _hop()
    plsc.subcore_barrier()

    _rescale(False)


_sc_propagate = pl.kernel(
    _sc_body,
    out_type=(
        jax.ShapeDtypeStruct((NPAD, HALF), jnp.float32),
        jax.ShapeDtypeStruct((NPAD, HALF), jnp.float32),
    ),
    mesh=plsc.VectorSubcoreMesh(core_axis_name="c", subcore_axis_name="s"),
    compiler_params=pltpu.CompilerParams(use_tc_tiling_on_sc=False),
    scratch_types=[
        pltpu.VMEM_SHARED((NPAD, HALF), jnp.float32),
        pltpu.VMEM_SHARED((NPAD, 16), jnp.float32),
        pltpu.VMEM((QRPT, HALF), jnp.float32),
        pltpu.VMEM((RPT, 16), jnp.float32),
        pltpu.VMEM((CH, 16), jnp.float32),
        pltpu.VMEM((2 * U, CH), jnp.int32),
        pltpu.VMEM((2 * U, CH), jnp.int32),
        pltpu.VMEM((TAIL,), jnp.int32),
        pltpu.VMEM((TAIL,), jnp.int32),
        [pltpu.VMEM((CH, HALF), jnp.float32)] * U,
        [pltpu.SemaphoreType.DMA] * U,
        [pltpu.SemaphoreType.DMA] * U,
        [pltpu.SemaphoreType.DMA] * U,
        pltpu.SemaphoreType.DMA,
        pltpu.SemaphoreType.DMA,
    ],
)


def _mm_body(f0_ref, f1_ref, w0_ref, w1_ref, b_ref, o_ref):
    o_ref[...] = (
        jnp.dot(f0_ref[...], w0_ref[...], preferred_element_type=jnp.float32)
        + jnp.dot(f1_ref[...], w1_ref[...], preferred_element_type=jnp.float32)
        + b_ref[...]
    )


BM = 1024

_tc_matmul = pl.pallas_call(
    _mm_body,
    grid=(NPAD // BM,),
    in_specs=[
        pl.BlockSpec((BM, HALF), lambda i: (i, 0)),
        pl.BlockSpec((BM, HALF), lambda i: (i, 0)),
        pl.BlockSpec((HALF, C), lambda i: (0, 0)),
        pl.BlockSpec((HALF, C), lambda i: (0, 0)),
        pl.BlockSpec((1, C), lambda i: (0, 0)),
    ],
    out_specs=pl.BlockSpec((BM, C), lambda i: (i, 0)),
    out_shape=jax.ShapeDtypeStruct((NPAD, C), jnp.float32),
)


@jax.jit
def kernel(g, h, e, snorm_n, snorm_e, W, b):
    f0, f1 = _sc_propagate(g, h)
    out = _tc_matmul(f0, f1, W[:HALF], W[HALF:], b.reshape(1, C))
    return out[:N]
